# baseline (speedup 1.0000x reference)
import functools

import jax
import jax.numpy as jnp
from jax.experimental import pallas as pl

_N = 512
_F = 6
_K = 6
_MD = 64
_SPP = 2


def _silu(v):
    return v * jax.nn.sigmoid(v)


def _mm(a, b):
    return jnp.dot(a, b, preferred_element_type=jnp.float32)


def _sample_body(feats, coors, lparams, hparams, diag):
    one11 = jnp.ones((1, 1), jnp.float32)

    def row_of(col):
        return jax.lax.dot_general(
            one11, col, (((1,), (1,)), ((), ())),
            preferred_element_type=jnp.float32)

    for (w1, b1, w2, b2, gw, gb, ln_g, ln_b, scale,
         cw1, cb1, cw2, cb2, nw1, nb1, nw2, nb2) in lparams:
        w1a, w1b, w1c = w1[0:_F], w1[_F:2 * _F], w1[2 * _F:2 * _F + 1]
        nw1a, nw1b = nw1[0:_F], nw1[_F:_F + _MD]

        n2 = jnp.sum(coors * coors, axis=1, keepdims=True)
        g = jax.lax.dot_general(
            coors, coors, (((1,), (1,)), ((), ())),
            preferred_element_type=jnp.float32)
        d = n2 + row_of(n2) - 2.0 * g
        dwork = jnp.where(diag, jnp.float32(jnp.inf), d)

        fc = jnp.concatenate([feats, coors], axis=1)
        fc_ks = [fc]
        dist_ks = [jnp.zeros((_N, 1), jnp.float32)]
        for _k in range(_K - 1):
            minv = jnp.min(dwork, axis=1, keepdims=True)
            sel = dwork == minv
            dwork = jnp.where(sel, jnp.float32(jnp.inf), dwork)
            fc_ks.append(_mm(sel.astype(jnp.float32), fc))
            dist_ks.append(minv)

        fc_all = jnp.concatenate(fc_ks, axis=0)
        dist = jnp.concatenate(dist_ks, axis=0)
        fj = fc_all[:, :_F]
        rel = jnp.concatenate([coors] * _K, axis=0) - fc_all[:, _F:]

        ei = _mm(feats, w1a) + b1
        h = _silu(jnp.concatenate([ei] * _K, axis=0)
                  + _mm(fj, w1b) + _mm(dist, w1c))
        mij = _silu(_mm(h, w2) + b2)
        gate = jax.nn.sigmoid(_mm(mij, gw) + gb)
        mij = mij * gate

        t = _silu(_mm(mij, cw1) + cb1)
        cw = jnp.clip(_mm(t, cw2) + cb2, -1.0, 1.0)

        norm = jnp.sqrt(jnp.sum(rel * rel, axis=1, keepdims=True))
        reln = rel / jnp.maximum(norm, 1e-8) * scale
        contrib = cw * reln

        cdelta = jnp.sum(contrib.reshape(_K, _N, 3), axis=0)
        m_i = jnp.sum(mij.reshape(_K, _N, _MD), axis=0)
        coors_new = cdelta + coors

        mu = jnp.mean(feats, axis=1, keepdims=True)
        var = jnp.mean((feats - mu) ** 2, axis=1, keepdims=True)
        nf = (feats - mu) / jnp.sqrt(var + 1e-5) * ln_g + ln_b

        h2 = _silu(_mm(nf, nw1a) + _mm(m_i, nw1b) + nb1)
        feats = _mm(h2, nw2) + nb2 + feats
        coors = coors_new

    mw1, mb1, mw2, mb2 = hparams
    zf = jnp.mean(feats, axis=0, keepdims=True)
    zh = jax.nn.relu(_mm(zf, mw1) + mb1)
    z = _mm(zh, mw2) + mb2
    z2 = jnp.concatenate([z[:, 0:12], z[:, 12:24]], axis=0)
    return jnp.concatenate(
        [z2, jnp.zeros((27, 12), jnp.float32)], axis=0)


def _egnn_body(*refs):
    x_ref, pos_ref = refs[0], refs[1]
    o_ref = refs[-1]
    vals = [r[...] for r in refs[2:-1]]
    lparams = [tuple(vals[0:17]), tuple(vals[17:34])]
    hparams = tuple(vals[34:38])

    iota = jax.lax.broadcasted_iota(jnp.int32, (_N, _N), 1)
    riota = jax.lax.broadcasted_iota(jnp.int32, (_N, _N), 0)
    diag = iota == riota

    for s in range(_SPP):
        z = _sample_body(x_ref[s], pos_ref[s], lparams, hparams, diag)
        o_ref[s] = z


def _full_spec(a):
    nd = a.ndim
    return pl.BlockSpec(a.shape, lambda i, _nd=nd: (0,) * _nd)


def kernel(x, pos, params):
    b = x.shape[0]
    f32 = jnp.float32

    pinputs = []
    for lp in params['layers']:
        pinputs += [
            lp['edge_w1'], lp['edge_b1'][None, :],
            lp['edge_w2'], lp['edge_b2'][None],
            lp['gate_w'], lp['gate_b'][None],
            lp['ln_g'][None], lp['ln_b'][None], lp['coors_scale'][None],
            lp['coors_w1'], lp['coors_b1'][None],
            lp['coors_w2'], lp['coors_b2'][None],
            lp['node_w1'], lp['node_b1'][None],
            lp['node_w2'], lp['node_b2'][None],
        ]
    pinputs += [params['mlp_w1'], params['mlp_b1'][None],
                params['mlp_w2'], params['mlp_b2'][None]]

    in_specs = [pl.BlockSpec((_SPP, _N, _F), lambda i: (i, 0, 0)),
                pl.BlockSpec((_SPP, _N, 3), lambda i: (i, 0, 0))]
    in_specs += [_full_spec(a) for a in pinputs]

    out = pl.pallas_call(
        _egnn_body,
        grid=(b // _SPP,),
        in_specs=in_specs,
        out_specs=pl.BlockSpec((_SPP, 29, 12), lambda i: (i, 0, 0)),
        out_shape=jax.ShapeDtypeStruct((b, 29, 12), f32),
    )(x, pos, *pinputs)

    return out


# speedup vs baseline: 1.0257x; 1.0257x over previous
import functools

import jax
import jax.numpy as jnp
from jax.experimental import pallas as pl

_N = 512
_F = 6
_K = 6
_MD = 64
_SPP = 2


def _silu(v):
    return v * jax.nn.sigmoid(v)


def _mm(a, b):
    return jnp.dot(a, b, preferred_element_type=jnp.float32)


def _sample_body(feats, coors, lparams, hparams, diag):
    one11 = jnp.ones((1, 1), jnp.float32)

    def row_of(col):
        return jax.lax.dot_general(
            one11, col, (((1,), (1,)), ((), ())),
            preferred_element_type=jnp.float32)

    for (w1, b1, w2, b2, gw, gb, ln_g, ln_b, scale,
         cw1, cb1, cw2, cb2, nw1, nb1, nw2, nb2) in lparams:
        w1a, w1b, w1c = w1[0:_F], w1[_F:2 * _F], w1[2 * _F:2 * _F + 1]
        nw1a, nw1b = nw1[0:_F], nw1[_F:_F + _MD]

        n2 = jnp.sum(coors * coors, axis=1, keepdims=True)
        g = jax.lax.dot_general(
            coors, coors, (((1,), (1,)), ((), ())),
            preferred_element_type=jnp.float32)
        d = n2 + row_of(n2) - 2.0 * g
        dwork = jnp.where(diag, jnp.float32(jnp.inf), d)

        fc = jnp.concatenate([feats, coors], axis=1)
        fc_ks = [fc]
        dist_ks = [jnp.zeros((_N, 1), jnp.float32)]
        for _k in range(_K - 1):
            minv = jnp.min(dwork, axis=1, keepdims=True)
            sel = dwork == minv
            if _k < _K - 2:
                dwork = jnp.where(sel, jnp.float32(jnp.inf), dwork)
            fc_ks.append(_mm(sel.astype(jnp.float32), fc))
            dist_ks.append(minv)

        fc_all = jnp.concatenate(fc_ks, axis=0)
        dist = jnp.concatenate(dist_ks, axis=0)
        fj = fc_all[:, :_F]
        rel = jnp.concatenate([coors] * _K, axis=0) - fc_all[:, _F:]

        ei = _mm(feats, w1a) + b1
        h = _silu(jnp.concatenate([ei] * _K, axis=0)
                  + _mm(fj, w1b) + _mm(dist, w1c))
        mij = _silu(_mm(h, w2) + b2)
        gate = jax.nn.sigmoid(_mm(mij, gw) + gb)
        mij = mij * gate

        t = _silu(_mm(mij, cw1) + cb1)
        cw = jnp.clip(_mm(t, cw2) + cb2, -1.0, 1.0)

        norm2 = jnp.sum(rel * rel, axis=1, keepdims=True)
        reln = rel * (scale * jax.lax.rsqrt(jnp.maximum(norm2, 1e-16)))
        contrib = cw * reln

        cdelta = jnp.sum(contrib.reshape(_K, _N, 3), axis=0)
        m_i = jnp.sum(mij.reshape(_K, _N, _MD), axis=0)
        coors_new = cdelta + coors

        mu = jnp.mean(feats, axis=1, keepdims=True)
        var = jnp.mean((feats - mu) ** 2, axis=1, keepdims=True)
        nf = (feats - mu) * jax.lax.rsqrt(var + 1e-5) * ln_g + ln_b

        h2 = _silu(_mm(nf, nw1a) + _mm(m_i, nw1b) + nb1)
        feats = _mm(h2, nw2) + nb2 + feats
        coors = coors_new

    mw1, mb1, mw2, mb2 = hparams
    zf = jnp.mean(feats, axis=0, keepdims=True)
    zh = jax.nn.relu(_mm(zf, mw1) + mb1)
    return _mm(zh, mw2) + mb2


def _egnn_body(*refs):
    x_ref, pos_ref = refs[0], refs[1]
    o_ref = refs[-1]
    vals = [r[...] for r in refs[2:-1]]
    lparams = [tuple(vals[0:17]), tuple(vals[17:34])]
    hparams = tuple(vals[34:38])

    iota = jax.lax.broadcasted_iota(jnp.int32, (_N, _N), 1)
    riota = jax.lax.broadcasted_iota(jnp.int32, (_N, _N), 0)
    diag = iota == riota

    for s in range(_SPP):
        z = _sample_body(x_ref[s], pos_ref[s], lparams, hparams, diag)
        o_ref[s] = z


def _full_spec(a):
    nd = a.ndim
    return pl.BlockSpec(a.shape, lambda i, _nd=nd: (0,) * _nd)


def kernel(x, pos, params):
    b = x.shape[0]
    f32 = jnp.float32

    pinputs = []
    for lp in params['layers']:
        pinputs += [
            lp['edge_w1'], lp['edge_b1'][None, :],
            lp['edge_w2'], lp['edge_b2'][None],
            lp['gate_w'], lp['gate_b'][None],
            lp['ln_g'][None], lp['ln_b'][None], lp['coors_scale'][None],
            lp['coors_w1'], lp['coors_b1'][None],
            lp['coors_w2'], lp['coors_b2'][None],
            lp['node_w1'], lp['node_b1'][None],
            lp['node_w2'], lp['node_b2'][None],
        ]
    pinputs += [params['mlp_w1'], params['mlp_b1'][None],
                params['mlp_w2'], params['mlp_b2'][None]]

    in_specs = [pl.BlockSpec((_SPP, _N, _F), lambda i: (i, 0, 0)),
                pl.BlockSpec((_SPP, _N, 3), lambda i: (i, 0, 0))]
    in_specs += [_full_spec(a) for a in pinputs]

    out = pl.pallas_call(
        _egnn_body,
        grid=(b // _SPP,),
        in_specs=in_specs,
        out_specs=pl.BlockSpec((_SPP, 1, 24), lambda i: (i, 0, 0)),
        out_shape=jax.ShapeDtypeStruct((b, 1, 24), f32),
    )(x, pos, *pinputs)

    z = out.reshape(b, 2, 12)
    return jnp.pad(z, ((0, 0), (0, 27), (0, 0)))


# speedup vs baseline: 1.0471x; 1.0210x over previous
import jax
import jax.numpy as jnp
from jax.experimental import pallas as pl

_N = 512
_F = 6
_K = 6
_MD = 64
_SPP = 2

def _silu(v):
    return v * jax.nn.sigmoid(v)

def _mm(a, b):
    return jnp.dot(a, b, preferred_element_type=jnp.float32)

def _sample_body(feats, coors, lparams, hparams, diag):
    one11 = jnp.ones((1, 1), jnp.float32)

    def row_of(col):
        return jax.lax.dot_general(
            one11, col, (((1,), (1,)), ((), ())),
            preferred_element_type=jnp.float32)

    for (w1, b1, w2, b2, gw, gb, ln_g, ln_b, scale,
         cw1, cb1, cw2, cb2, nw1, nb1, nw2, nb2) in lparams:
        w1a, w1b, w1c = w1[0:_F], w1[_F:2 * _F], w1[2 * _F:2 * _F + 1]
        nw1a, nw1b = nw1[0:_F], nw1[_F:_F + _MD]

        n2 = jnp.sum(coors * coors, axis=1, keepdims=True)
        g = jax.lax.dot_general(
            coors, coors, (((1,), (1,)), ((), ())),
            preferred_element_type=jnp.float32)
        d = n2 + row_of(n2) - 2.0 * g
        dwork = jnp.where(diag, jnp.float32(jnp.inf), d)

        fc = jnp.concatenate([feats, coors], axis=1)
        fc_ks = [fc]
        dist_ks = [jnp.zeros((_N, 1), jnp.float32)]
        for _k in range(_K - 1):
            minv = jnp.min(dwork, axis=1, keepdims=True)
            sel = dwork == minv
            if _k < _K - 2:
                dwork = jnp.where(sel, jnp.float32(jnp.inf), dwork)
            fc_ks.append(_mm(sel.astype(jnp.float32), fc))
            dist_ks.append(minv)

        dist = jnp.concatenate(dist_ks, axis=0)
        rel = jnp.concatenate([coors] * _K, axis=0) - jnp.concatenate(
            [f[:, _F:] for f in fc_ks], axis=0)

        eh = _F * 2 + 1
        oh = eh * 2
        fjP = jnp.concatenate([f[:, :_F] for f in fc_ks], axis=1)
        distP = jnp.concatenate(dist_ks, axis=1)
        def _bd(w, ow):
            rows = []
            for k in range(_K):
                parts = []
                if k:
                    parts.append(jnp.zeros((w.shape[0], ow * k), jnp.float32))
                parts.append(w)
                if k < _K - 1:
                    parts.append(
                        jnp.zeros((w.shape[0], ow * (_K - 1 - k)), jnp.float32))
                rows.append(jnp.concatenate(parts, axis=1) if len(parts) > 1
                            else parts[0])
            return jnp.concatenate(rows, axis=0)

        w1b_bd = _bd(w1b, oh)
        w1c_bd = _bd(w1c, oh)
        w2_bd = _bd(w2, _MD)
        gw_bd = _bd(gw, 1)
        ei = _mm(feats, w1a) + b1
        hP = _silu(jnp.concatenate([ei] * _K, axis=1)
                   + _mm(fjP, w1b_bd) + _mm(distP, w1c_bd))
        mijP = _silu(_mm(hP, w2_bd)
                     + jnp.concatenate([b2] * _K, axis=1))
        gateP = jax.nn.sigmoid(_mm(mijP, gw_bd) + gb)
        ones_row = jnp.ones((1, _MD), jnp.float32)
        gexp = jnp.concatenate(
            [_mm(gateP[:, k:k + 1], ones_row) for k in range(_K)],
            axis=1)
        mijP = mijP * gexp

        mij_ks = [mijP[:, _MD * k:_MD * (k + 1)] for k in range(_K)]
        mij = jnp.concatenate(mij_ks, axis=0)

        t = _silu(_mm(mij, cw1) + cb1)
        cw = jnp.clip(_mm(t, cw2) + cb2, -1.0, 1.0)

        norm2 = jnp.sum(rel * rel, axis=1, keepdims=True)
        reln = rel * (scale * jax.lax.rsqrt(jnp.maximum(norm2, 1e-16)))
        contrib = cw * reln

        cdelta = jnp.sum(contrib.reshape(_K, _N, 3), axis=0)
        m_i = mij_ks[0]
        for mk in mij_ks[1:]:
            m_i = m_i + mk
        coors_new = cdelta + coors

        mu = jnp.mean(feats, axis=1, keepdims=True)
        var = jnp.mean((feats - mu) ** 2, axis=1, keepdims=True)
        nf = (feats - mu) * jax.lax.rsqrt(var + 1e-5) * ln_g + ln_b

        h2 = _silu(_mm(nf, nw1a) + _mm(m_i, nw1b) + nb1)
        feats = _mm(h2, nw2) + nb2 + feats
        coors = coors_new

    mw1, mb1, mw2, mb2 = hparams
    zf = jnp.mean(feats, axis=0, keepdims=True)
    zh = jax.nn.relu(_mm(zf, mw1) + mb1)
    return _mm(zh, mw2) + mb2

def _egnn_body(*refs):
    x_ref, pos_ref = refs[0], refs[1]
    o_ref = refs[-1]
    vals = [r[...] for r in refs[2:-1]]
    lparams = [tuple(vals[0:17]), tuple(vals[17:34])]
    hparams = tuple(vals[34:38])

    iota = jax.lax.broadcasted_iota(jnp.int32, (_N, _N), 1)
    riota = jax.lax.broadcasted_iota(jnp.int32, (_N, _N), 0)
    diag = iota == riota

    for s in range(_SPP):
        z = _sample_body(x_ref[s], pos_ref[s], lparams, hparams, diag)
        o_ref[s] = z

def _full_spec(a):
    nd = a.ndim
    return pl.BlockSpec(a.shape, lambda i, _nd=nd: (0,) * _nd)

def kernel(x, pos, params):
    b = x.shape[0]
    f32 = jnp.float32

    pinputs = []
    for lp in params['layers']:
        pinputs += [
            lp['edge_w1'], lp['edge_b1'][None, :],
            lp['edge_w2'], lp['edge_b2'][None],
            lp['gate_w'], lp['gate_b'][None],
            lp['ln_g'][None], lp['ln_b'][None], lp['coors_scale'][None],
            lp['coors_w1'], lp['coors_b1'][None],
            lp['coors_w2'], lp['coors_b2'][None],
            lp['node_w1'], lp['node_b1'][None],
            lp['node_w2'], lp['node_b2'][None],
        ]
    pinputs += [params['mlp_w1'], params['mlp_b1'][None],
                params['mlp_w2'], params['mlp_b2'][None]]

    in_specs = [pl.BlockSpec((_SPP, _N, _F), lambda i: (i, 0, 0)),
                pl.BlockSpec((_SPP, _N, 3), lambda i: (i, 0, 0))]
    in_specs += [_full_spec(a) for a in pinputs]

    out = pl.pallas_call(
        _egnn_body,
        grid=(b // _SPP,),
        in_specs=in_specs,
        out_specs=pl.BlockSpec((_SPP, 1, 24), lambda i: (i, 0, 0)),
        out_shape=jax.ShapeDtypeStruct((b, 1, 24), f32),
    )(x, pos, *pinputs)

    z = out.reshape(b, 2, 12)
    return jnp.pad(z, ((0, 0), (0, 27), (0, 0)))
